# baseline (speedup 1.0000x reference)
import functools

import jax
import jax.numpy as jnp
from jax import lax
from jax.experimental import pallas as pl
from jax.experimental.pallas import tpu as pltpu
from jax.experimental.pallas import tpu_sc as plsc

_NUM_CLASSES = 100000
_D = 128
_B = 1024
_EPS = 1e-6

_BK = 4000
_NKB = _NUM_CLASSES // _BK
_BIG = 2**30


_NCH = 10
_CH = _BK // _NCH
_S = 40


def _nearest_body(q_ref, keys_ref, out_ref, s_ref, w_ref, acc_ref, iacc_ref):
    kb = pl.program_id(0)

    @pl.when(kb == 0)
    def _init():
        acc_ref[...] = jnp.full((_S, _B), -jnp.inf, jnp.float32)
        iacc_ref[...] = jnp.zeros((_S, _B), jnp.int32)

    qb = q_ref[...].astype(jnp.bfloat16)

    def _emit_chunk(c):
        keys_c = keys_ref[pl.ds(c * _CH, _CH), :]
        dots = lax.dot_general(
            keys_c.astype(jnp.bfloat16), qb, (((1,), (1,)), ((), ())),
            preferred_element_type=jnp.float32)
        knsq = jnp.sum(keys_c * keys_c, axis=1, keepdims=True)
        s_ref[pl.ds(c * _CH, _CH), :] = dots
        w_ref[pl.ds(c * _CH, _CH), :] = 1.0 / jnp.sqrt(knsq)

    def _scan_chunk(c, carry):
        acc, iacc = carry
        for j in range(_CH // _S):
            r0 = c * _CH + j * _S
            strip = s_ref[pl.ds(r0, _S), :] * w_ref[pl.ds(r0, _S), :]
            cmp = strip > acc
            acc = jnp.maximum(strip, acc)
            iacc = jnp.where(cmp, kb * (_BK // _S) + r0 // _S, iacc)
        return acc, iacc

    _emit_chunk(0)
    carry = (acc_ref[...], iacc_ref[...])
    for c in range(1, _NCH):
        _emit_chunk(c)
        carry = _scan_chunk(c - 1, carry)
    carry = _scan_chunk(_NCH - 1, carry)
    acc_ref[...], iacc_ref[...] = carry

    @pl.when(kb == _NKB - 1)
    def _done():
        acc = acc_ref[...]
        kidx = iacc_ref[...] * _S + lax.broadcasted_iota(
            jnp.int32, (_S, _B), 0)
        gmax = jnp.max(acc, axis=0, keepdims=True)
        cand = jnp.where(acc == gmax, kidx, _BIG)
        out_ref[...] = jnp.min(cand, axis=0, keepdims=True)


def _nearest_tc(query_embeddings, template_table, interpret=False):
    return pl.pallas_call(
        _nearest_body,
        grid=(_NKB,),
        in_specs=[
            pl.BlockSpec((_B, _D), lambda kb: (0, 0)),
            pl.BlockSpec((_BK, _D), lambda kb: (kb, 0)),
        ],
        out_specs=pl.BlockSpec((1, _B), lambda kb: (0, 0)),
        out_shape=jax.ShapeDtypeStruct((1, _B), jnp.int32),
        scratch_shapes=[
            pltpu.VMEM((_BK, _B), jnp.float32),
            pltpu.VMEM((_BK, 1), jnp.float32),
            pltpu.VMEM((_S, _B), jnp.float32),
            pltpu.VMEM((_S, _B), jnp.int32),
        ],
        compiler_params=pltpu.CompilerParams(
            dimension_semantics=("arbitrary",)),
        interpret=interpret,
    )(query_embeddings, template_table)


_NC = 2
_NS = 16
_L = 16
_NW = _NC * _NS
_BPW = _B // _NW


@functools.cache
def _make_gather():
    @functools.partial(
        pl.kernel,
        out_type=jax.ShapeDtypeStruct((_B, _D), jnp.float32),
        mesh=plsc.VectorSubcoreMesh(core_axis_name="c", subcore_axis_name="s"),
        scratch_types=[
            pltpu.VMEM((_BPW,), jnp.int32),
            pltpu.VMEM((_BPW,), jnp.int32),
            pltpu.VMEM((_BPW,), jnp.int32),
            pltpu.VMEM((_BPW, _D), jnp.float32),
            pltpu.SemaphoreType.DMA,
        ],
    )
    def _gather_kernel(table_hbm, ev_hbm, near_hbm, out_hbm,
                       ev_v, near_v, idx_v, rows_v, sem):
        wid = lax.axis_index("s") * _NC + lax.axis_index("c")
        base = wid * _BPW
        pltpu.sync_copy(ev_hbm.at[pl.ds(base, _BPW)], ev_v)
        pltpu.sync_copy(near_hbm.at[pl.ds(base, _BPW)], near_v)
        for i in range(_BPW // _L):
            sl = pl.ds(i * _L, _L)
            ev = ev_v[sl]
            idx_v[sl] = jnp.where(ev > _NUM_CLASSES, near_v[sl], ev)
        pltpu.async_copy(table_hbm.at[idx_v], rows_v, sem).wait()
        pltpu.sync_copy(rows_v, out_hbm.at[pl.ds(base, _BPW)])

    return _gather_kernel


def kernel(event_ids, query_embeddings, template_table):
    nearest = _nearest_tc(query_embeddings, template_table)
    return _make_gather()(template_table, event_ids, nearest.reshape(_B))


# speedup vs baseline: 1.0544x; 1.0544x over previous
import functools

import jax
import jax.numpy as jnp
from jax import lax
from jax.experimental import pallas as pl
from jax.experimental.pallas import tpu as pltpu
from jax.experimental.pallas import tpu_sc as plsc

_NUM_CLASSES = 100000
_D = 128
_B = 1024
_EPS = 1e-6

_BK = 10000
_NKB = _NUM_CLASSES // _BK
_BIG = 2**30


_NCH = 25
_CH = _BK // _NCH
_S = 16


def _nearest_body(q_ref, keys_ref, out_ref, s_ref, w_ref, acc_ref, iacc_ref):
    kb = pl.program_id(0)

    @pl.when(kb == 0)
    def _init():
        acc_ref[...] = jnp.full((_S, _B), -jnp.inf, jnp.float32)
        iacc_ref[...] = jnp.zeros((_S, _B), jnp.int32)

    qb = q_ref[...].astype(jnp.bfloat16)

    def _emit_chunk(c):
        keys_c = keys_ref[pl.ds(c * _CH, _CH), :]
        dots = lax.dot_general(
            keys_c.astype(jnp.bfloat16), qb, (((1,), (1,)), ((), ())),
            preferred_element_type=jnp.float32)
        knsq = jnp.sum(keys_c * keys_c, axis=1, keepdims=True)
        s_ref[pl.ds(c * _CH, _CH), :] = dots
        w_ref[pl.ds(c * _CH, _CH), :] = 1.0 / jnp.sqrt(knsq)

    def _scan_chunk(c, carry):
        acc, iacc = carry
        for j in range(_CH // _S):
            r0 = c * _CH + j * _S
            strip = s_ref[pl.ds(r0, _S), :] * w_ref[pl.ds(r0, _S), :]
            cmp = strip > acc
            acc = jnp.maximum(strip, acc)
            iacc = jnp.where(cmp, kb * (_BK // _S) + r0 // _S, iacc)
        return acc, iacc

    _emit_chunk(0)
    carry = (acc_ref[...], iacc_ref[...])
    for c in range(1, _NCH):
        _emit_chunk(c)
        carry = _scan_chunk(c - 1, carry)
    carry = _scan_chunk(_NCH - 1, carry)
    acc_ref[...], iacc_ref[...] = carry

    @pl.when(kb == _NKB - 1)
    def _done():
        acc = acc_ref[...]
        kidx = iacc_ref[...] * _S + lax.broadcasted_iota(
            jnp.int32, (_S, _B), 0)
        gmax = jnp.max(acc, axis=0, keepdims=True)
        cand = jnp.where(acc == gmax, kidx, _BIG)
        out_ref[...] = jnp.min(cand, axis=0, keepdims=True)


def _nearest_tc(query_embeddings, template_table, interpret=False):
    return pl.pallas_call(
        _nearest_body,
        grid=(_NKB,),
        in_specs=[
            pl.BlockSpec((_B, _D), lambda kb: (0, 0)),
            pl.BlockSpec((_BK, _D), lambda kb: (kb, 0)),
        ],
        out_specs=pl.BlockSpec((1, _B), lambda kb: (0, 0)),
        out_shape=jax.ShapeDtypeStruct((1, _B), jnp.int32),
        scratch_shapes=[
            pltpu.VMEM((_BK, _B), jnp.float32),
            pltpu.VMEM((_BK, 1), jnp.float32),
            pltpu.VMEM((_S, _B), jnp.float32),
            pltpu.VMEM((_S, _B), jnp.int32),
        ],
        compiler_params=pltpu.CompilerParams(
            dimension_semantics=("arbitrary",)),
        interpret=interpret,
    )(query_embeddings, template_table)


_NC = 2
_NS = 16
_L = 16
_NW = _NC * _NS
_BPW = _B // _NW


@functools.cache
def _make_gather():
    @functools.partial(
        pl.kernel,
        out_type=jax.ShapeDtypeStruct((_B, _D), jnp.float32),
        mesh=plsc.VectorSubcoreMesh(core_axis_name="c", subcore_axis_name="s"),
        scratch_types=[
            pltpu.VMEM((_BPW,), jnp.int32),
            pltpu.VMEM((_BPW,), jnp.int32),
            pltpu.VMEM((_BPW,), jnp.int32),
            pltpu.VMEM((_BPW, _D), jnp.float32),
            pltpu.SemaphoreType.DMA,
        ],
    )
    def _gather_kernel(table_hbm, ev_hbm, near_hbm, out_hbm,
                       ev_v, near_v, idx_v, rows_v, sem):
        wid = lax.axis_index("s") * _NC + lax.axis_index("c")
        base = wid * _BPW
        pltpu.sync_copy(ev_hbm.at[pl.ds(base, _BPW)], ev_v)
        pltpu.sync_copy(near_hbm.at[pl.ds(base, _BPW)], near_v)
        for i in range(_BPW // _L):
            sl = pl.ds(i * _L, _L)
            ev = ev_v[sl]
            idx_v[sl] = jnp.where(ev > _NUM_CLASSES, near_v[sl], ev)
        pltpu.async_copy(table_hbm.at[idx_v], rows_v, sem).wait()
        pltpu.sync_copy(rows_v, out_hbm.at[pl.ds(base, _BPW)])

    return _gather_kernel


def kernel(event_ids, query_embeddings, template_table):
    nearest = _nearest_tc(query_embeddings, template_table)
    return _make_gather()(template_table, event_ids, nearest.reshape(_B))


# speedup vs baseline: 1.0724x; 1.0171x over previous
import functools

import jax
import jax.numpy as jnp
from jax import lax
from jax.experimental import pallas as pl
from jax.experimental.pallas import tpu as pltpu
from jax.experimental.pallas import tpu_sc as plsc

_NUM_CLASSES = 100000
_D = 128
_B = 1024
_EPS = 1e-6

_BK = 10000
_NKB = _NUM_CLASSES // _BK
_BIG = 2**30


_NCH = 5
_CH = _BK // _NCH
_S = 16


def _nearest_body(q_ref, keys_ref, out_ref, s_ref, w_ref, acc_ref, iacc_ref):
    kb = pl.program_id(0)

    @pl.when(kb == 0)
    def _init():
        acc_ref[...] = jnp.full((_S, _B), -jnp.inf, jnp.float32)
        iacc_ref[...] = jnp.zeros((_S, _B), jnp.int32)

    qb = q_ref[...].astype(jnp.bfloat16)

    def _emit_chunk(c):
        keys_c = keys_ref[pl.ds(c * _CH, _CH), :]
        dots = lax.dot_general(
            keys_c.astype(jnp.bfloat16), qb, (((1,), (1,)), ((), ())),
            preferred_element_type=jnp.float32)
        knsq = jnp.sum(keys_c * keys_c, axis=1, keepdims=True)
        s_ref[pl.ds(c * _CH, _CH), :] = dots
        w_ref[pl.ds(c * _CH, _CH), :] = 1.0 / jnp.sqrt(knsq)

    def _scan_chunk(c, carry):
        acc, iacc = carry
        for j in range(_CH // _S):
            r0 = c * _CH + j * _S
            strip = s_ref[pl.ds(r0, _S), :] * w_ref[pl.ds(r0, _S), :]
            cmp = strip > acc
            acc = jnp.maximum(strip, acc)
            iacc = jnp.where(cmp, kb * (_BK // _S) + r0 // _S, iacc)
        return acc, iacc

    _emit_chunk(0)
    carry = (acc_ref[...], iacc_ref[...])
    for c in range(1, _NCH):
        _emit_chunk(c)
        carry = _scan_chunk(c - 1, carry)
    carry = _scan_chunk(_NCH - 1, carry)
    acc_ref[...], iacc_ref[...] = carry

    @pl.when(kb == _NKB - 1)
    def _done():
        acc = acc_ref[...]
        kidx = iacc_ref[...] * _S + lax.broadcasted_iota(
            jnp.int32, (_S, _B), 0)
        gmax = jnp.max(acc, axis=0, keepdims=True)
        cand = jnp.where(acc == gmax, kidx, _BIG)
        out_ref[...] = jnp.min(cand, axis=0, keepdims=True)


def _nearest_tc(query_embeddings, template_table, interpret=False):
    return pl.pallas_call(
        _nearest_body,
        grid=(_NKB,),
        in_specs=[
            pl.BlockSpec((_B, _D), lambda kb: (0, 0)),
            pl.BlockSpec((_BK, _D), lambda kb: (kb, 0)),
        ],
        out_specs=pl.BlockSpec((1, _B), lambda kb: (0, 0)),
        out_shape=jax.ShapeDtypeStruct((1, _B), jnp.int32),
        scratch_shapes=[
            pltpu.VMEM((_BK, _B), jnp.float32),
            pltpu.VMEM((_BK, 1), jnp.float32),
            pltpu.VMEM((_S, _B), jnp.float32),
            pltpu.VMEM((_S, _B), jnp.int32),
        ],
        compiler_params=pltpu.CompilerParams(
            dimension_semantics=("arbitrary",)),
        interpret=interpret,
    )(query_embeddings, template_table)


_NC = 2
_NS = 16
_L = 16
_NW = _NC * _NS
_BPW = _B // _NW


@functools.cache
def _make_gather():
    @functools.partial(
        pl.kernel,
        out_type=jax.ShapeDtypeStruct((_B, _D), jnp.float32),
        mesh=plsc.VectorSubcoreMesh(core_axis_name="c", subcore_axis_name="s"),
        scratch_types=[
            pltpu.VMEM((_BPW,), jnp.int32),
            pltpu.VMEM((_BPW,), jnp.int32),
            pltpu.VMEM((_BPW,), jnp.int32),
            pltpu.VMEM((_BPW, _D), jnp.float32),
            pltpu.SemaphoreType.DMA,
        ],
    )
    def _gather_kernel(table_hbm, ev_hbm, near_hbm, out_hbm,
                       ev_v, near_v, idx_v, rows_v, sem):
        wid = lax.axis_index("s") * _NC + lax.axis_index("c")
        base = wid * _BPW
        pltpu.sync_copy(ev_hbm.at[pl.ds(base, _BPW)], ev_v)
        pltpu.sync_copy(near_hbm.at[pl.ds(base, _BPW)], near_v)
        for i in range(_BPW // _L):
            sl = pl.ds(i * _L, _L)
            ev = ev_v[sl]
            idx_v[sl] = jnp.where(ev > _NUM_CLASSES, near_v[sl], ev)
        pltpu.async_copy(table_hbm.at[idx_v], rows_v, sem).wait()
        pltpu.sync_copy(rows_v, out_hbm.at[pl.ds(base, _BPW)])

    return _gather_kernel


def kernel(event_ids, query_embeddings, template_table):
    nearest = _nearest_tc(query_embeddings, template_table)
    return _make_gather()(template_table, event_ids, nearest.reshape(_B))
